# baseline (speedup 1.0000x reference)
import functools

import jax
import jax.numpy as jnp
from jax import lax
from jax.experimental import pallas as pl
from jax.experimental.pallas import tpu as pltpu
from jax.experimental.pallas import tpu_sc as plsc

DIM = 64
SCALE = 8.0

NC, NS = 2, 16
NW = NC * NS
TR = 128
NB = 4
CHUNK = NB * TR


def _emb_sc(idx2d, table, *, n_transfers):
    t_per_w = n_transfers // NW
    s_per_w = t_per_w // NB
    rows_per_w = t_per_w * TR
    b_total = n_transfers * TR

    mesh = plsc.VectorSubcoreMesh(core_axis_name="c", subcore_axis_name="s")

    @functools.partial(
        pl.kernel,
        out_type=jax.ShapeDtypeStruct((b_total, DIM), jnp.float32),
        mesh=mesh,
        scratch_types=[
            pltpu.VMEM((t_per_w, TR), jnp.int32),
            pltpu.VMEM((2, CHUNK, DIM), jnp.float32),
            pltpu.SemaphoreType.DMA,
            pltpu.SemaphoreType.DMA,
            pltpu.SemaphoreType.DMA,
            pltpu.SemaphoreType.DMA,
        ],
        compiler_params=pltpu.CompilerParams(use_tc_tiling_on_sc=False),
    )
    def k(idx_hbm, table_hbm, out_hbm, idx_v, rows_v, g0, g1, o0, o1):
        gsem = (g0, g1)
        osem = (o0, o1)
        wid = lax.axis_index("s") * NC + lax.axis_index("c")
        tbase = wid * t_per_w
        rbase = wid * rows_per_w

        pltpu.sync_copy(idx_hbm.at[pl.ds(tbase, t_per_w)], idx_v)

        def stage(slot, chunk):
            for j in range(NB):
                pltpu.async_copy(
                    table_hbm.at[idx_v.at[chunk * NB + j]],
                    rows_v.at[slot, pl.ds(j * TR, TR)],
                    gsem[slot],
                )

        def drain(slot, sems):
            pltpu.make_async_copy(
                table_hbm.at[pl.ds(0, CHUNK)], rows_v.at[slot], sems[slot]
            ).wait()

        def scale(slot):
            @plsc.parallel_loop(0, CHUNK, unroll=8)
            def _(r):
                for c in range(DIM // 16):
                    sl = (slot, r, pl.ds(c * 16, 16))
                    rows_v[sl] = rows_v[sl] * SCALE

        def flush(slot, chunk):
            pltpu.async_copy(
                rows_v.at[slot],
                out_hbm.at[pl.ds(rbase + chunk * CHUNK, CHUNK)],
                osem[slot],
            )

        stage(0, 0)
        stage(1, 1)

        @pl.loop(0, s_per_w // 2 - 1)
        def _(i):
            c0 = 2 * i
            drain(0, gsem)
            scale(0)
            flush(0, c0)
            drain(1, gsem)
            scale(1)
            flush(1, c0 + 1)
            drain(0, osem)
            stage(0, c0 + 2)
            drain(1, osem)
            stage(1, c0 + 3)

        drain(0, gsem)
        scale(0)
        flush(0, s_per_w - 2)
        drain(1, gsem)
        scale(1)
        flush(1, s_per_w - 1)
        drain(0, osem)
        drain(1, osem)

    return k(idx2d, table)


def kernel(x, table):
    b0, b1 = x.shape
    b = b0 * b1
    idx2d = x.reshape(b // TR, TR).astype(jnp.int32)
    out = _emb_sc(idx2d, table, n_transfers=b // TR)
    return out.reshape(b0, b1, DIM)
